# baseline (speedup 1.0000x reference)
import dataclasses
import functools

import jax
import jax.numpy as jnp
from jax import lax
from jax.experimental import pallas as pl
from jax.experimental.pallas import tpu as pltpu
from jax.experimental.pallas import tpu_sc as plsc

N = 10000
NP = 10240
E = 320000
BLK = 128
CHB = 8
NCH = 20
NB = NCH * CHB
EP = 16 * NB * BLK
DIN = 128
DQ = 64
DHID = 256
DOUT = 128
RPT = NP // 16
NBUF = 4

_mesh = plsc.VectorSubcoreMesh(core_axis_name="c", subcore_axis_name="s")

_cp = dataclasses.replace(pltpu.CompilerParams(),
                          needs_layout_passes=False,
                          use_tc_tiling_on_sc=False)


def _make_sc_pass(qbases, with_deg, chb, nch):
    out_type = [jax.ShapeDtypeStruct((2 * len(qbases), NP, DQ),
                                     jnp.float32)]
    scratch = [
        pltpu.VMEM((chb, BLK), jnp.int32),
        pltpu.VMEM((chb, BLK), jnp.int32),
    ]
    scratch += [pltpu.VMEM((BLK, DQ), jnp.float32) for _ in range(NBUF)]
    scratch += [
        pltpu.VMEM_SHARED((NP, DQ), jnp.float32),
        pltpu.VMEM_SHARED((NP, DQ), jnp.float32),
    ]
    scratch += [pltpu.SemaphoreType.DMA] * (2 * NBUF)
    if with_deg:
        out_type.append(jax.ShapeDtypeStruct((32, NP), jnp.float32))
        scratch.insert(2 + NBUF, pltpu.VMEM((NP,), jnp.float32))

    @functools.partial(pl.kernel, mesh=_mesh, out_type=out_type,
                       scratch_types=scratch, compiler_params=_cp)
    def k(tab_hbm, src_hbm, dst_hbm, agg_hbm, *rest):
        rest = list(rest)
        deg_hbm = rest.pop(0) if with_deg else None
        src_v, dst_v = rest[0], rest[1]
        rows = rest[2:2 + NBUF]
        deg_v = rest[2 + NBUF] if with_deg else None
        base = 2 + NBUF + (1 if with_deg else 0)
        tab_sh, acc_sh = rest[base], rest[base + 1]
        gsems = rest[base + 2:base + 2 + NBUF]
        ssems = rest[base + 2 + NBUF:base + 2 + 2 * NBUF]
        c = lax.axis_index("c")
        s = lax.axis_index("s")

        ones16 = jnp.ones((16,), jnp.float32)
        half = nch // 2

        def issue_g(b):
            pltpu.async_copy(tab_sh.at[src_v.at[b]],
                             rows[b % NBUF], gsems[b % NBUF])

        def wait_g(b):
            pltpu.make_async_copy(tab_sh.at[src_v.at[b]],
                                  rows[b % NBUF], gsems[b % NBUF]).wait()

        def issue_s(b):
            pltpu.async_copy(rows[b % NBUF], acc_sh.at[dst_v.at[b]],
                             ssems[b % NBUF], add=True)

        def wait_s(b):
            pltpu.make_async_copy(rows[b % NBUF], acc_sh.at[dst_v.at[0]],
                                  ssems[b % NBUF]).wait()

        if with_deg:
            @pl.loop(0, NP // 16)
            def _(i):
                deg_v[pl.ds(i * 16, 16)] = jnp.zeros((16,), jnp.float32)

        for rnd, qbase in enumerate(qbases):
            pltpu.sync_copy(tab_hbm.at[qbase + c, pl.ds(s * RPT, RPT)],
                            tab_sh.at[pl.ds(s * RPT, RPT)])

            @pl.loop(0, BLK)
            def _(i):
                @pl.loop(0, DQ // 16)
                def _(j):
                    rows[0][i, pl.ds(j * 16, 16)] = jnp.zeros(
                        (16,), jnp.float32)

            @pl.loop(0, RPT // BLK)
            def _(r):
                pltpu.sync_copy(rows[0],
                                acc_sh.at[pl.ds(s * RPT + r * BLK, BLK)])

            plsc.subcore_barrier()

            @pl.loop(0, nch)
            def _(ch):
                pltpu.sync_copy(src_hbm.at[s, ch], src_v)
                pltpu.sync_copy(dst_hbm.at[s, ch], dst_v)
                for i in range(min(2, chb)):
                    issue_g(i)
                for b in range(chb):
                    if b + 2 < chb:
                        if b >= 2:
                            wait_s(b - 2)
                        issue_g(b + 2)
                    wait_g(b)
                    issue_s(b)
                    if with_deg:
                        mine = jnp.where(c == 0, ch < half, ch >= half)

                        @pl.when(mine)
                        def _():
                            @pl.loop(0, BLK // 16)
                            def _(j):
                                idx = dst_v[b, pl.ds(j * 16, 16)]
                                plsc.addupdate_scatter(deg_v, [idx], ones16)
                for b in range(max(0, chb - NBUF), chb):
                    wait_s(b)

            plsc.subcore_barrier()
            pltpu.sync_copy(acc_sh.at[pl.ds(s * RPT, RPT)],
                            agg_hbm.at[2 * rnd + c, pl.ds(s * RPT, RPT)])

        if with_deg:
            pltpu.sync_copy(deg_v, deg_hbm.at[c * 16 + s])

    return k


_sc_pass1 = _make_sc_pass([0], True, CHB, NCH)
_sc_pass2 = _make_sc_pass([0, 2], False, CHB, NCH)


def _tc_root1(x, wr1, b1_2d):
    def body(x_ref, wr_ref, b_ref, out_ref):
        out_ref[...] = (jnp.dot(x_ref[...], wr_ref[...],
                                preferred_element_type=jnp.float32)
                        + b_ref[...])

    return pl.pallas_call(
        body,
        out_shape=jax.ShapeDtypeStruct((N, DHID), jnp.float32),
    )(x, wr1, b1_2d)


def _tc1_combine(aggp, degp, root1, wl1a, wl1b):
    def body(agg_ref, deg_ref, root_ref, wla_ref, wlb_ref,
             hq_ref, inv_ref):
        degs = jnp.sum(deg_ref[...], axis=0)
        inv = 1.0 / jnp.maximum(degs[:N], 1.0)
        invc = inv.reshape(N, 1)
        m0 = agg_ref[0, :N, :] * invc
        m1 = agg_ref[1, :N, :] * invc
        pre = (jnp.dot(m0, wla_ref[...], preferred_element_type=jnp.float32)
               + jnp.dot(m1, wlb_ref[...], preferred_element_type=jnp.float32)
               + root_ref[...])
        h = jnp.where(pre > 0, pre, jnp.exp(pre) - 1.0)
        for q in range(4):
            hq_ref[q, :N, :] = h[:, q * DQ:(q + 1) * DQ]
        inv_ref[...] = invc

    return pl.pallas_call(
        body,
        out_shape=[
            jax.ShapeDtypeStruct((4, NP, DQ), jnp.float32),
            jax.ShapeDtypeStruct((N, 1), jnp.float32),
        ],
    )(aggp, degp, root1, wl1a, wl1b)


def _tc_root2(hq, wrmu, wrls, bmu_2d, bls_2d):
    R = 2000

    def body(hq_ref, wrmu_ref, wrls_ref, bmu_ref, bls_ref,
             rmu_ref, rls_ref):
        accm = bmu_ref[...]
        accl = bls_ref[...]
        for q in range(4):
            hqv = hq_ref[q]
            accm = accm + jnp.dot(hqv, wrmu_ref[pl.ds(q * DQ, DQ), :],
                                  preferred_element_type=jnp.float32)
            accl = accl + jnp.dot(hqv, wrls_ref[pl.ds(q * DQ, DQ), :],
                                  preferred_element_type=jnp.float32)
        rmu_ref[...] = accm
        rls_ref[...] = accl

    return pl.pallas_call(
        body,
        grid=(N // R,),
        in_specs=[
            pl.BlockSpec((4, R, DQ), lambda i: (0, i, 0)),
            pl.BlockSpec((DHID, DOUT), lambda i: (0, 0)),
            pl.BlockSpec((DHID, DOUT), lambda i: (0, 0)),
            pl.BlockSpec((1, DOUT), lambda i: (0, 0)),
            pl.BlockSpec((1, DOUT), lambda i: (0, 0)),
        ],
        out_specs=[
            pl.BlockSpec((R, DOUT), lambda i: (i, 0)),
            pl.BlockSpec((R, DOUT), lambda i: (i, 0)),
        ],
        out_shape=[
            jax.ShapeDtypeStruct((N, DOUT), jnp.float32),
            jax.ShapeDtypeStruct((N, DOUT), jnp.float32),
        ],
    )(hq, wrmu, wrls, bmu_2d, bls_2d)


def _tc2_combine(agg2, inv_deg, rmu, rls, wlmu, wlls):
    R = 2000

    def body(agg2_ref, inv_ref, rmu_ref, rls_ref,
             wlmu_ref, wlls_ref, mu_ref, ls_ref):
        invc = inv_ref[...]
        accm = rmu_ref[...]
        accl = rls_ref[...]
        for q in range(4):
            m = agg2_ref[q] * invc
            accm = accm + jnp.dot(m, wlmu_ref[pl.ds(q * DQ, DQ), :],
                                  preferred_element_type=jnp.float32)
            accl = accl + jnp.dot(m, wlls_ref[pl.ds(q * DQ, DQ), :],
                                  preferred_element_type=jnp.float32)
        mu_ref[...] = accm
        ls_ref[...] = accl

    return pl.pallas_call(
        body,
        grid=(N // R,),
        in_specs=[
            pl.BlockSpec((4, R, DQ), lambda i: (0, i, 0)),
            pl.BlockSpec((R, 1), lambda i: (i, 0)),
            pl.BlockSpec((R, DOUT), lambda i: (i, 0)),
            pl.BlockSpec((R, DOUT), lambda i: (i, 0)),
            pl.BlockSpec((DHID, DOUT), lambda i: (0, 0)),
            pl.BlockSpec((DHID, DOUT), lambda i: (0, 0)),
        ],
        out_specs=[
            pl.BlockSpec((R, DOUT), lambda i: (i, 0)),
            pl.BlockSpec((R, DOUT), lambda i: (i, 0)),
        ],
        out_shape=[
            jax.ShapeDtypeStruct((N, DOUT), jnp.float32),
            jax.ShapeDtypeStruct((N, DOUT), jnp.float32),
        ],
    )(agg2, inv_deg, rmu, rls, wlmu, wlls)


def kernel(x, edge_index, W_l1, W_r1, b1, W_lmu, W_rmu, b_mu,
           W_lls, W_rls, b_ls):
    src = edge_index[0]
    dst = edge_index[1]
    pad = EP - E
    src_p = jnp.concatenate([src, jnp.zeros((pad,), jnp.int32)])
    dst_p = jnp.concatenate([dst, jnp.full((pad,), N, jnp.int32)])
    src4 = src_p.reshape(16, NCH, CHB, BLK)
    dst4 = dst_p.reshape(16, NCH, CHB, BLK)
    xp = jnp.pad(x, ((0, NP - N), (0, 0))).reshape(NP, 2, DQ).transpose(1, 0, 2)

    aggp, degp = _sc_pass1(xp, src4, dst4)
    root1 = _tc_root1(x, W_r1, b1.reshape(1, -1))
    hq, inv_deg = _tc1_combine(aggp, degp, root1, W_l1[:DQ], W_l1[DQ:])
    agg2, = _sc_pass2(hq, src4, dst4)
    rmu, rls = _tc_root2(hq, W_rmu, W_rls, b_mu.reshape(1, -1),
                         b_ls.reshape(1, -1))
    mu, logstd = _tc2_combine(agg2, inv_deg, rmu, rls, W_lmu, W_lls)
    return (mu, logstd)
